# baseline (speedup 1.0000x reference)
import functools

import jax
import jax.numpy as jnp
from jax import lax
from jax.experimental import pallas as pl
from jax.experimental.pallas import tpu as pltpu
from jax.experimental.pallas import tpu_sc as plsc

_LANES = 16
_PACK = 8
_TCOL = 8192


def _tc_transpose_pack(tab_cm, nf, vocab, dim):
    g = -(-vocab // _TCOL)
    kb = _TCOL // _PACK

    def body(x_ref, eye_ref, out_ref, s0, s1):
        x = x_ref[...]
        eye = eye_ref[...]
        s0[...] = jax.lax.dot_general(
            x[:dim], eye, (((0,), (0,)), ((), ())),
            preferred_element_type=jnp.float32)
        s1[...] = jax.lax.dot_general(
            x[dim:], eye, (((0,), (0,)), ((), ())),
            preferred_element_type=jnp.float32)
        for r in range(_PACK):
            sl = pl.Slice(r, kb, _PACK)
            out_ref[0, :, r * dim:(r + 1) * dim] = s0[sl, :]
            out_ref[1, :, r * dim:(r + 1) * dim] = s1[sl, :]

    return pl.pallas_call(
        body,
        grid=(g,),
        in_specs=[
            pl.BlockSpec((nf * dim, _TCOL), lambda i: (0, i)),
            pl.BlockSpec((dim, dim), lambda i: (0, 0)),
        ],
        out_specs=pl.BlockSpec((nf, kb, _PACK * dim), lambda i: (0, i, 0)),
        out_shape=jax.ShapeDtypeStruct(
            (nf, vocab // _PACK, _PACK * dim), jnp.float32),
        scratch_shapes=[
            pltpu.VMEM((_TCOL, dim), jnp.float32),
            pltpu.VMEM((_TCOL, dim), jnp.float32),
        ],
        compiler_params=pltpu.CompilerParams(
            fuse_transposed_lhs_in_matmul=True),
    )(tab_cm, jnp.eye(dim, dtype=jnp.float32))


def _sc_gather(tabp, ids_flat, vocab, dim):
    n_rows = ids_flat.shape[0]
    info = plsc.get_sparse_core_info()
    nc, ns = info.num_cores, info.num_subcores
    nw = nc * ns
    b_per_w = n_rows // nw
    half = b_per_w // 2
    feat_rows = n_rows // 2
    lines_per_feat = vocab // _PACK
    mesh = plsc.VectorSubcoreMesh(core_axis_name="c", subcore_axis_name="s")

    @functools.partial(
        pl.kernel,
        mesh=mesh,
        out_type=jax.ShapeDtypeStruct((n_rows, dim), jnp.float32),
        scratch_types=[
            pltpu.VMEM((b_per_w,), jnp.int32),
            pltpu.VMEM((half,), jnp.int32),
            pltpu.VMEM((half, _PACK * dim), jnp.float32),
            pltpu.VMEM((b_per_w, dim), jnp.float32),
            pltpu.SemaphoreType.DMA,
        ],
        compiler_params=pltpu.CompilerParams(
            use_tc_tiling_on_sc=False, needs_layout_passes=False),
    )
    def gather_k(tab_hbm, idx_hbm, out_hbm, idx_v, line_v, rows_v, emb_v, sem):
        wid = lax.axis_index("s") * nc + lax.axis_index("c")
        f = wid % 2
        j = wid // 2
        base = f * feat_rows + j * b_per_w
        pltpu.sync_copy(idx_hbm.at[pl.ds(base, b_per_w)], idx_v)
        lane = jax.lax.iota(jnp.int32, _LANES)

        for ch in range(2):

            def mk_lines(i, c):
                sl = pl.ds(ch * half + i * _LANES, _LANES)
                dsl = pl.ds(i * _LANES, _LANES)
                line_v[dsl] = (
                    lax.shift_right_logical(idx_v[sl], 3)
                    + f * lines_per_feat)
                return c

            lax.fori_loop(0, half // _LANES, mk_lines, 0)
            pltpu.async_copy(tab_hbm.at[line_v], rows_v, sem).wait()

            def sel(p, c):
                sl = pl.ds(ch * half + p * _LANES, _LANES)
                colb = jnp.bitwise_and(idx_v[sl], 7) * dim
                prow = lane + p * _LANES
                for dcomp in range(dim):
                    vals = plsc.load_gather(rows_v, [prow, colb + dcomp])
                    plsc.store_scatter(
                        emb_v,
                        [prow + ch * half,
                         jnp.broadcast_to(jnp.int32(dcomp), (_LANES,))],
                        vals)
                return c

            lax.fori_loop(0, half // _LANES, sel, 0)
        pltpu.sync_copy(emb_v, out_hbm.at[pl.ds(base, b_per_w)])

    return gather_k(tabp, ids_flat)


def _tc_mlp(gath, w1, b1, w2, b2, w3, b3, interpret=False):
    n_rows, dim = gath.shape
    batch = n_rows // 2
    r = 1024
    g = batch // r
    h1 = w1.shape[1]
    h2 = w2.shape[1]

    def body(x0, x1, w1r, b1r, w2r, b2r, w3r, b3r, out):
        x = jnp.concatenate([x0[...], x1[...]], axis=0)
        h = jnp.dot(x, w1r[...], preferred_element_type=jnp.float32)
        h = jnp.maximum(h + b1r[...], 0.0)
        h = jnp.dot(h, w2r[...], preferred_element_type=jnp.float32)
        h = jnp.maximum(h + b2r[...], 0.0)
        p = jnp.sum(h * w3r[...], axis=1) + b3r[0, 0]
        out[0, 0, :] = p[:r] + p[r:]

    out = pl.pallas_call(
        body,
        grid=(g,),
        in_specs=[
            pl.BlockSpec((r, dim), lambda i: (i, 0)),
            pl.BlockSpec((r, dim), lambda i: (i + g, 0)),
            pl.BlockSpec((dim, h1), lambda i: (0, 0)),
            pl.BlockSpec((1, h1), lambda i: (0, 0)),
            pl.BlockSpec((h1, h2), lambda i: (0, 0)),
            pl.BlockSpec((1, h2), lambda i: (0, 0)),
            pl.BlockSpec((1, h2), lambda i: (0, 0)),
            pl.BlockSpec((1, 1), lambda i: (0, 0)),
        ],
        out_specs=pl.BlockSpec((1, 1, r), lambda i: (i, 0, 0)),
        out_shape=jax.ShapeDtypeStruct((g, 1, r), jnp.float32),
        interpret=interpret,
    )(gath, gath, w1, b1.reshape(1, h1), w2, b2.reshape(1, h2),
      w3.reshape(1, h2), b3.reshape(1, 1))
    return out.reshape(batch)


def kernel(kjt_ids, tables, W1, b1, W2, b2, W3, b3):
    nf, vocab, dim = tables.shape
    ids_flat = kjt_ids.reshape(-1).astype(jnp.int32)
    tab_cm = tables.transpose(0, 2, 1).reshape(nf * dim, vocab)
    tabp = _tc_transpose_pack(tab_cm, nf, vocab, dim)
    tabp2 = tabp.reshape(nf * (vocab // _PACK), _PACK * dim)
    gath = _sc_gather(tabp2, ids_flat, vocab, dim)
    return _tc_mlp(gath, W1, b1, W2, b2, W3, b3)


# speedup vs baseline: 1.0815x; 1.0815x over previous
import functools

import jax
import jax.numpy as jnp
from jax import lax
from jax.experimental import pallas as pl
from jax.experimental.pallas import tpu as pltpu
from jax.experimental.pallas import tpu_sc as plsc

_LANES = 16
_PACK = 8
_TCOL = 8192


def _tc_transpose_pack(tab_cm, nf, vocab, dim):
    g = -(-vocab // _TCOL)
    kb = _TCOL // _PACK

    line = _PACK * dim

    def body(x_ref, eye_ref, e_ref, out_ref, s0, s1):
        x = x_ref[...]
        eye = eye_ref[...]
        s0[...] = jax.lax.dot_general(
            x[:dim], eye, (((0,), (0,)), ((), ())),
            preferred_element_type=jnp.float32)
        s1[...] = jax.lax.dot_general(
            x[dim:], eye, (((0,), (0,)), ((), ())),
            preferred_element_type=jnp.float32)
        for f, s in ((0, s0), (1, s1)):
            acc = jnp.zeros((kb, line), jnp.float32)
            for r in range(_PACK):
                yr = s[pl.Slice(r, kb, _PACK), :]
                er = e_ref[:, r * line:(r + 1) * line]
                acc = acc + jnp.dot(
                    yr, er, preferred_element_type=jnp.float32)
            out_ref[f] = acc

    cols = jnp.arange(_PACK * line)
    e_mat = (
        (cols[None, :] % line)
        == (cols[None, :] // line) * dim + jnp.arange(dim)[:, None]
    ).astype(jnp.float32)

    return pl.pallas_call(
        body,
        grid=(g,),
        in_specs=[
            pl.BlockSpec((nf * dim, _TCOL), lambda i: (0, i)),
            pl.BlockSpec((dim, dim), lambda i: (0, 0)),
            pl.BlockSpec((dim, _PACK * line), lambda i: (0, 0)),
        ],
        out_specs=pl.BlockSpec((nf, kb, line), lambda i: (0, i, 0)),
        out_shape=jax.ShapeDtypeStruct(
            (nf, vocab // _PACK, line), jnp.float32),
        scratch_shapes=[
            pltpu.VMEM((_TCOL, dim), jnp.float32),
            pltpu.VMEM((_TCOL, dim), jnp.float32),
        ],
        compiler_params=pltpu.CompilerParams(
            fuse_transposed_lhs_in_matmul=True),
    )(tab_cm, jnp.eye(dim, dtype=jnp.float32), e_mat)


def _sc_gather(tabp, ids_flat, vocab, dim):
    n_rows = ids_flat.shape[0]
    info = plsc.get_sparse_core_info()
    nc, ns = info.num_cores, info.num_subcores
    nw = nc * ns
    b_per_w = n_rows // nw
    half = b_per_w // 2
    feat_rows = n_rows // 2
    lines_per_feat = vocab // _PACK
    mesh = plsc.VectorSubcoreMesh(core_axis_name="c", subcore_axis_name="s")

    @functools.partial(
        pl.kernel,
        mesh=mesh,
        out_type=jax.ShapeDtypeStruct((n_rows, dim), jnp.float32),
        scratch_types=[
            pltpu.VMEM((b_per_w,), jnp.int32),
            pltpu.VMEM((half,), jnp.int32),
            pltpu.VMEM((half, _PACK * dim), jnp.float32),
            pltpu.VMEM((b_per_w, dim), jnp.float32),
            pltpu.SemaphoreType.DMA,
        ],
        compiler_params=pltpu.CompilerParams(
            use_tc_tiling_on_sc=False, needs_layout_passes=False),
    )
    def gather_k(tab_hbm, idx_hbm, out_hbm, idx_v, line_v, rows_v, emb_v, sem):
        wid = lax.axis_index("s") * nc + lax.axis_index("c")
        f = wid % 2
        j = wid // 2
        base = f * feat_rows + j * b_per_w
        pltpu.sync_copy(idx_hbm.at[pl.ds(base, b_per_w)], idx_v)
        lane = jax.lax.iota(jnp.int32, _LANES)

        for ch in range(2):

            def mk_lines(i, c):
                sl = pl.ds(ch * half + i * _LANES, _LANES)
                dsl = pl.ds(i * _LANES, _LANES)
                line_v[dsl] = (
                    lax.shift_right_logical(idx_v[sl], 3)
                    + f * lines_per_feat)
                return c

            lax.fori_loop(0, half // _LANES, mk_lines, 0)
            pltpu.async_copy(tab_hbm.at[line_v], rows_v, sem).wait()

            def sel(p, c):
                sl = pl.ds(ch * half + p * _LANES, _LANES)
                colb = jnp.bitwise_and(idx_v[sl], 7) * dim
                prow = lane + p * _LANES
                for dcomp in range(dim):
                    vals = plsc.load_gather(rows_v, [prow, colb + dcomp])
                    plsc.store_scatter(
                        emb_v,
                        [prow + ch * half,
                         jnp.broadcast_to(jnp.int32(dcomp), (_LANES,))],
                        vals)
                return c

            lax.fori_loop(0, half // _LANES, sel, 0)
        pltpu.sync_copy(emb_v, out_hbm.at[pl.ds(base, b_per_w)])

    return gather_k(tabp, ids_flat)


def _tc_mlp(gath, w1, b1, w2, b2, w3, b3, interpret=False):
    n_rows, dim = gath.shape
    batch = n_rows // 2
    r = 1024
    g = batch // r
    h1 = w1.shape[1]
    h2 = w2.shape[1]

    def body(x0, x1, w1r, b1r, w2r, b2r, w3r, b3r, out):
        x = jnp.concatenate([x0[...], x1[...]], axis=0)
        h = jnp.dot(x, w1r[...], preferred_element_type=jnp.float32)
        h = jnp.maximum(h + b1r[...], 0.0)
        h = jnp.dot(h, w2r[...], preferred_element_type=jnp.float32)
        h = jnp.maximum(h + b2r[...], 0.0)
        p = jnp.sum(h * w3r[...], axis=1) + b3r[0, 0]
        out[0, 0, :] = p[:r] + p[r:]

    out = pl.pallas_call(
        body,
        grid=(g,),
        in_specs=[
            pl.BlockSpec((r, dim), lambda i: (i, 0)),
            pl.BlockSpec((r, dim), lambda i: (i + g, 0)),
            pl.BlockSpec((dim, h1), lambda i: (0, 0)),
            pl.BlockSpec((1, h1), lambda i: (0, 0)),
            pl.BlockSpec((h1, h2), lambda i: (0, 0)),
            pl.BlockSpec((1, h2), lambda i: (0, 0)),
            pl.BlockSpec((1, h2), lambda i: (0, 0)),
            pl.BlockSpec((1, 1), lambda i: (0, 0)),
        ],
        out_specs=pl.BlockSpec((1, 1, r), lambda i: (i, 0, 0)),
        out_shape=jax.ShapeDtypeStruct((g, 1, r), jnp.float32),
        interpret=interpret,
    )(gath, gath, w1, b1.reshape(1, h1), w2, b2.reshape(1, h2),
      w3.reshape(1, h2), b3.reshape(1, 1))
    return out.reshape(batch)


def kernel(kjt_ids, tables, W1, b1, W2, b2, W3, b3):
    nf, vocab, dim = tables.shape
    ids_flat = kjt_ids.reshape(-1).astype(jnp.int32)
    tab_cm = tables.transpose(0, 2, 1).reshape(nf * dim, vocab)
    tabp = _tc_transpose_pack(tab_cm, nf, vocab, dim)
    tabp2 = tabp.reshape(nf * (vocab // _PACK), _PACK * dim)
    gath = _sc_gather(tabp2, ids_flat, vocab, dim)
    return _tc_mlp(gath, W1, b1, W2, b2, W3, b3)


# speedup vs baseline: 1.7457x; 1.6141x over previous
import functools

import jax
import jax.numpy as jnp
from jax import lax
from jax.experimental import pallas as pl
from jax.experimental.pallas import tpu as pltpu
from jax.experimental.pallas import tpu_sc as plsc

_LANES = 16
_PACK = 8
_TCOL = 8192


def _tc_transpose_pack(tab_cm, nf, vocab, dim):
    g = -(-vocab // _TCOL)
    kb = _TCOL // _PACK

    line = _PACK * dim
    nd = nf * dim

    def body(x_ref, eye_ref, e_ref, out_ref, s01):
        s01[...] = jax.lax.dot_general(
            x_ref[...], eye_ref[...], (((0,), (0,)), ((), ())),
            preferred_element_type=jnp.float32)
        for f in range(nf):
            acc = jnp.zeros((kb, line), jnp.float32)
            for r in range(_PACK):
                yr = s01[pl.Slice(r, kb, _PACK), :]
                er = e_ref[:, (f * _PACK + r) * line:
                           (f * _PACK + r + 1) * line]
                acc = acc + jnp.dot(
                    yr, er, preferred_element_type=jnp.float32)
            out_ref[f] = acc

    cols = jnp.arange(nf * _PACK * line)
    fi = cols // (_PACK * line)
    rc = cols % (_PACK * line)
    r_of = rc // line
    e_of = rc % line
    rows = jnp.arange(nd)[:, None]
    e_mat = (
        ((e_of[None, :] - r_of[None, :] * dim) == (rows - fi[None, :] * dim))
        & (rows // dim == fi[None, :])
    ).astype(jnp.float32)

    return pl.pallas_call(
        body,
        grid=(g,),
        in_specs=[
            pl.BlockSpec((nd, _TCOL), lambda i: (0, i)),
            pl.BlockSpec((nd, nd), lambda i: (0, 0)),
            pl.BlockSpec((nd, nf * _PACK * line), lambda i: (0, 0)),
        ],
        out_specs=pl.BlockSpec((nf, kb, line), lambda i: (0, i, 0)),
        out_shape=jax.ShapeDtypeStruct(
            (nf, vocab // _PACK, line), jnp.float32),
        scratch_shapes=[
            pltpu.VMEM((_TCOL, nd), jnp.float32),
        ],
        compiler_params=pltpu.CompilerParams(
            fuse_transposed_lhs_in_matmul=True),
    )(tab_cm, jnp.eye(nd, dtype=jnp.float32), e_mat)


def _sc_gather(tabp, ids_flat, vocab, dim):
    n_rows = ids_flat.shape[0]
    info = plsc.get_sparse_core_info()
    nc, ns = info.num_cores, info.num_subcores
    nw = nc * ns
    b_per_w = n_rows // nw
    half = b_per_w // 2
    feat_rows = n_rows // 2
    lines_per_feat = vocab // _PACK
    mesh = plsc.VectorSubcoreMesh(core_axis_name="c", subcore_axis_name="s")

    @functools.partial(
        pl.kernel,
        mesh=mesh,
        out_type=jax.ShapeDtypeStruct((n_rows, dim), jnp.float32),
        scratch_types=[
            pltpu.VMEM((b_per_w,), jnp.int32),
            pltpu.VMEM((half,), jnp.int32),
            pltpu.VMEM((half, _PACK * dim), jnp.float32),
            pltpu.VMEM((b_per_w, dim), jnp.float32),
            pltpu.SemaphoreType.DMA,
        ],
        compiler_params=pltpu.CompilerParams(
            use_tc_tiling_on_sc=False, needs_layout_passes=False),
    )
    def gather_k(tab_hbm, idx_hbm, out_hbm, idx_v, line_v, rows_v, emb_v, sem):
        wid = lax.axis_index("s") * nc + lax.axis_index("c")
        f = wid % 2
        j = wid // 2
        base = f * feat_rows + j * b_per_w
        pltpu.sync_copy(idx_hbm.at[pl.ds(base, b_per_w)], idx_v)
        lane = jax.lax.iota(jnp.int32, _LANES)

        for ch in range(2):

            def mk_lines(i, c):
                sl = pl.ds(ch * half + i * _LANES, _LANES)
                dsl = pl.ds(i * _LANES, _LANES)
                line_v[dsl] = (
                    lax.shift_right_logical(idx_v[sl], 3)
                    + f * lines_per_feat)
                return c

            lax.fori_loop(0, half // _LANES, mk_lines, 0)
            pltpu.async_copy(tab_hbm.at[line_v], rows_v, sem).wait()

            def sel(p, c):
                sl = pl.ds(ch * half + p * _LANES, _LANES)
                colb = jnp.bitwise_and(idx_v[sl], 7) * dim
                prow = lane + p * _LANES
                for dcomp in range(dim):
                    vals = plsc.load_gather(rows_v, [prow, colb + dcomp])
                    plsc.store_scatter(
                        emb_v,
                        [prow + ch * half,
                         jnp.broadcast_to(jnp.int32(dcomp), (_LANES,))],
                        vals)
                return c

            lax.fori_loop(0, half // _LANES, sel, 0)
        pltpu.sync_copy(emb_v, out_hbm.at[pl.ds(base, b_per_w)])

    return gather_k(tabp, ids_flat)


def _tc_mlp(gath, w1, b1, w2, b2, w3, b3, interpret=False):
    n_rows, dim = gath.shape
    batch = n_rows // 2
    r = 1024
    g = batch // r
    h1 = w1.shape[1]
    h2 = w2.shape[1]

    def body(x0, x1, w1r, b1r, w2r, b2r, w3r, b3r, out):
        x = jnp.concatenate([x0[...], x1[...]], axis=0)
        h = jnp.dot(x, w1r[...], preferred_element_type=jnp.float32)
        h = jnp.maximum(h + b1r[...], 0.0)
        h = jnp.dot(h, w2r[...], preferred_element_type=jnp.float32)
        h = jnp.maximum(h + b2r[...], 0.0)
        p = jnp.sum(h * w3r[...], axis=1) + b3r[0, 0]
        out[0, 0, :] = p[:r] + p[r:]

    out = pl.pallas_call(
        body,
        grid=(g,),
        in_specs=[
            pl.BlockSpec((r, dim), lambda i: (i, 0)),
            pl.BlockSpec((r, dim), lambda i: (i + g, 0)),
            pl.BlockSpec((dim, h1), lambda i: (0, 0)),
            pl.BlockSpec((1, h1), lambda i: (0, 0)),
            pl.BlockSpec((h1, h2), lambda i: (0, 0)),
            pl.BlockSpec((1, h2), lambda i: (0, 0)),
            pl.BlockSpec((1, h2), lambda i: (0, 0)),
            pl.BlockSpec((1, 1), lambda i: (0, 0)),
        ],
        out_specs=pl.BlockSpec((1, 1, r), lambda i: (i, 0, 0)),
        out_shape=jax.ShapeDtypeStruct((g, 1, r), jnp.float32),
        interpret=interpret,
    )(gath, gath, w1, b1.reshape(1, h1), w2, b2.reshape(1, h2),
      w3.reshape(1, h2), b3.reshape(1, 1))
    return out.reshape(batch)


def kernel(kjt_ids, tables, W1, b1, W2, b2, W3, b3):
    nf, vocab, dim = tables.shape
    ids_flat = kjt_ids.reshape(-1).astype(jnp.int32)
    tab_cm = tables.transpose(0, 2, 1).reshape(nf * dim, vocab)
    tabp = _tc_transpose_pack(tab_cm, nf, vocab, dim)
    tabp2 = tabp.reshape(nf * (vocab // _PACK), _PACK * dim)
    gath = _sc_gather(tabp2, ids_flat, vocab, dim)
    return _tc_mlp(gath, W1, b1, W2, b2, W3, b3)


# speedup vs baseline: 1.8173x; 1.0410x over previous
import functools

import jax
import jax.numpy as jnp
from jax import lax
from jax.experimental import pallas as pl
from jax.experimental.pallas import tpu as pltpu
from jax.experimental.pallas import tpu_sc as plsc

_LANES = 16
_PACK = 8
_TCOL = 16384


def _tc_transpose_pack(tab_cm, nf, vocab, dim):
    g = -(-vocab // _TCOL)
    kb = _TCOL // _PACK

    line = _PACK * dim
    nd = nf * dim

    def body(x_ref, eye_ref, e_ref, out_ref, s01):
        s01[...] = jax.lax.dot_general(
            x_ref[...], eye_ref[...], (((0,), (0,)), ((), ())),
            preferred_element_type=jnp.float32)
        for f in range(nf):
            acc = jnp.zeros((kb, line), jnp.float32)
            for r in range(_PACK):
                yr = s01[pl.Slice(r, kb, _PACK), :]
                er = e_ref[:, (f * _PACK + r) * line:
                           (f * _PACK + r + 1) * line]
                acc = acc + jnp.dot(
                    yr, er, preferred_element_type=jnp.float32)
            out_ref[f] = acc

    cols = jnp.arange(nf * _PACK * line)
    fi = cols // (_PACK * line)
    rc = cols % (_PACK * line)
    r_of = rc // line
    e_of = rc % line
    rows = jnp.arange(nd)[:, None]
    e_mat = (
        ((e_of[None, :] - r_of[None, :] * dim) == (rows - fi[None, :] * dim))
        & (rows // dim == fi[None, :])
    ).astype(jnp.float32)

    return pl.pallas_call(
        body,
        grid=(g,),
        in_specs=[
            pl.BlockSpec((nd, _TCOL), lambda i: (0, i)),
            pl.BlockSpec((nd, nd), lambda i: (0, 0)),
            pl.BlockSpec((nd, nf * _PACK * line), lambda i: (0, 0)),
        ],
        out_specs=pl.BlockSpec((nf, kb, line), lambda i: (0, i, 0)),
        out_shape=jax.ShapeDtypeStruct(
            (nf, vocab // _PACK, line), jnp.float32),
        scratch_shapes=[
            pltpu.VMEM((_TCOL, nd), jnp.float32),
        ],
        compiler_params=pltpu.CompilerParams(
            fuse_transposed_lhs_in_matmul=True),
    )(tab_cm, jnp.eye(nd, dtype=jnp.float32), e_mat)


def _sc_gather(tabp, ids_flat, vocab, dim):
    n_rows = ids_flat.shape[0]
    info = plsc.get_sparse_core_info()
    nc, ns = info.num_cores, info.num_subcores
    nw = nc * ns
    b_per_w = n_rows // nw
    half = b_per_w // 2
    feat_rows = n_rows // 2
    lines_per_feat = vocab // _PACK
    mesh = plsc.VectorSubcoreMesh(core_axis_name="c", subcore_axis_name="s")

    @functools.partial(
        pl.kernel,
        mesh=mesh,
        out_type=jax.ShapeDtypeStruct((n_rows, dim), jnp.float32),
        scratch_types=[
            pltpu.VMEM((b_per_w,), jnp.int32),
            pltpu.VMEM((half,), jnp.int32),
            pltpu.VMEM((half, _PACK * dim), jnp.float32),
            pltpu.VMEM((b_per_w, dim), jnp.float32),
            pltpu.SemaphoreType.DMA,
        ],
        compiler_params=pltpu.CompilerParams(
            use_tc_tiling_on_sc=False, needs_layout_passes=False),
    )
    def gather_k(tab_hbm, idx_hbm, out_hbm, idx_v, line_v, rows_v, emb_v, sem):
        wid = lax.axis_index("s") * nc + lax.axis_index("c")
        f = wid % 2
        j = wid // 2
        base = f * feat_rows + j * b_per_w
        pltpu.sync_copy(idx_hbm.at[pl.ds(base, b_per_w)], idx_v)
        lane = jax.lax.iota(jnp.int32, _LANES)

        for ch in range(2):

            def mk_lines(i, c):
                sl = pl.ds(ch * half + i * _LANES, _LANES)
                dsl = pl.ds(i * _LANES, _LANES)
                line_v[dsl] = (
                    lax.shift_right_logical(idx_v[sl], 3)
                    + f * lines_per_feat)
                return c

            lax.fori_loop(0, half // _LANES, mk_lines, 0)
            pltpu.async_copy(tab_hbm.at[line_v], rows_v, sem).wait()

            def sel(p, c):
                sl = pl.ds(ch * half + p * _LANES, _LANES)
                colb = jnp.bitwise_and(idx_v[sl], 7) * dim
                prow = lane + p * _LANES
                for dcomp in range(dim):
                    vals = plsc.load_gather(rows_v, [prow, colb + dcomp])
                    plsc.store_scatter(
                        emb_v,
                        [prow + ch * half,
                         jnp.broadcast_to(jnp.int32(dcomp), (_LANES,))],
                        vals)
                return c

            lax.fori_loop(0, half // _LANES, sel, 0)
        pltpu.sync_copy(emb_v, out_hbm.at[pl.ds(base, b_per_w)])

    return gather_k(tabp, ids_flat)


def _tc_mlp(gath, w1, b1, w2, b2, w3, b3, interpret=False):
    n_rows, dim = gath.shape
    batch = n_rows // 2
    r = 1024
    g = batch // r
    h1 = w1.shape[1]
    h2 = w2.shape[1]

    def body(x0, x1, w1r, b1r, w2r, b2r, w3r, b3r, out):
        x = jnp.concatenate([x0[...], x1[...]], axis=0)
        h = jnp.dot(x, w1r[...], preferred_element_type=jnp.float32)
        h = jnp.maximum(h + b1r[...], 0.0)
        h = jnp.dot(h, w2r[...], preferred_element_type=jnp.float32)
        h = jnp.maximum(h + b2r[...], 0.0)
        p = jnp.sum(h * w3r[...], axis=1) + b3r[0, 0]
        out[0, 0, :] = p[:r] + p[r:]

    out = pl.pallas_call(
        body,
        grid=(g,),
        in_specs=[
            pl.BlockSpec((r, dim), lambda i: (i, 0)),
            pl.BlockSpec((r, dim), lambda i: (i + g, 0)),
            pl.BlockSpec((dim, h1), lambda i: (0, 0)),
            pl.BlockSpec((1, h1), lambda i: (0, 0)),
            pl.BlockSpec((h1, h2), lambda i: (0, 0)),
            pl.BlockSpec((1, h2), lambda i: (0, 0)),
            pl.BlockSpec((1, h2), lambda i: (0, 0)),
            pl.BlockSpec((1, 1), lambda i: (0, 0)),
        ],
        out_specs=pl.BlockSpec((1, 1, r), lambda i: (i, 0, 0)),
        out_shape=jax.ShapeDtypeStruct((g, 1, r), jnp.float32),
        interpret=interpret,
    )(gath, gath, w1, b1.reshape(1, h1), w2, b2.reshape(1, h2),
      w3.reshape(1, h2), b3.reshape(1, 1))
    return out.reshape(batch)


def kernel(kjt_ids, tables, W1, b1, W2, b2, W3, b3):
    nf, vocab, dim = tables.shape
    ids_flat = kjt_ids.reshape(-1).astype(jnp.int32)
    tab_cm = tables.transpose(0, 2, 1).reshape(nf * dim, vocab)
    tabp = _tc_transpose_pack(tab_cm, nf, vocab, dim)
    tabp2 = tabp.reshape(nf * (vocab // _PACK), _PACK * dim)
    gath = _sc_gather(tabp2, ids_flat, vocab, dim)
    return _tc_mlp(gath, W1, b1, W2, b2, W3, b3)
